# Initial kernel scaffold; baseline (speedup 1.0000x reference)
import jax
import jax.numpy as jnp
from jax.experimental import pallas as pl


def kernel(x, W_enc, b_enc, W_dec, b_dec):
    raise NotImplementedError("write your pallas kernel here")


# speedup vs baseline: 1.3069x; 1.3069x over previous
import functools

import jax
import jax.numpy as jnp
from jax import lax
from jax.experimental import pallas as pl
from jax.experimental.pallas import tpu as pltpu
from jax.experimental.pallas import tpu_sc as plsc

B, D, K = 8192, 512, 1024

BM = 512
NB = B // BM
KB = K // NB

NC, NS = 2, 16
NW = NC * NS
BPW = B // NW
CHUNK = 128
NCH = BPW // CHUNK


def _argmax_body(x_ref, we_ref, be_ref, wd_ref, bd_ref, idx_ref, tab_ref):
    logits = jnp.dot(x_ref[...], we_ref[...],
                     preferred_element_type=jnp.float32)
    logits = logits + be_ref[...]
    m = jnp.max(logits, axis=1, keepdims=True)
    ii = lax.broadcasted_iota(jnp.int32, logits.shape, 1)
    idx = jnp.min(jnp.where(logits == m, ii, K), axis=1)
    idx_ref[...] = idx.reshape(idx_ref.shape)
    tab_ref[...] = wd_ref[...] + bd_ref[...]


_argmax_call = pl.pallas_call(
    _argmax_body,
    grid=(NB,),
    in_specs=[
        pl.BlockSpec((BM, D), lambda i: (i, 0)),
        pl.BlockSpec((D, K), lambda i: (0, 0)),
        pl.BlockSpec((1, K), lambda i: (0, 0)),
        pl.BlockSpec((KB, D), lambda i: (i, 0)),
        pl.BlockSpec((1, D), lambda i: (0, 0)),
    ],
    out_specs=[
        pl.BlockSpec((1, 1, BM), lambda i: (i, 0, 0)),
        pl.BlockSpec((KB, D), lambda i: (i, 0)),
    ],
    out_shape=[
        jax.ShapeDtypeStruct((NB, 1, BM), jnp.int32),
        jax.ShapeDtypeStruct((K, D), jnp.float32),
    ],
)


@functools.cache
def _sc_gather_call():
    @functools.partial(
        pl.kernel,
        out_type=jax.ShapeDtypeStruct((B, D), jnp.float32),
        mesh=plsc.VectorSubcoreMesh(core_axis_name="c", subcore_axis_name="s",
                                    num_cores=NC, num_subcores=NS),
        scratch_types=[
            pltpu.VMEM((CHUNK,), jnp.int32),
            pltpu.VMEM((CHUNK, D), jnp.float32),
            pltpu.SemaphoreType.DMA,
        ],
    )
    def _sc_gather(tab_hbm, idx_hbm, out_hbm, idx_v, rows_v, sem):
        wid = lax.axis_index("s") * NC + lax.axis_index("c")
        base = wid * BPW
        for c in range(NCH):
            off = base + c * CHUNK
            pltpu.sync_copy(idx_hbm.at[pl.ds(off, CHUNK)], idx_v)
            pltpu.async_copy(tab_hbm.at[idx_v], rows_v, sem).wait()
            pltpu.sync_copy(rows_v, out_hbm.at[pl.ds(off, CHUNK)])

    return _sc_gather


def kernel(x, W_enc, b_enc, W_dec, b_dec):
    idx3, tab = _argmax_call(x, W_enc, b_enc.reshape(1, K),
                             W_dec, b_dec.reshape(1, D))
    return _sc_gather_call()(tab, idx3.reshape(B))


# speedup vs baseline: 1.5229x; 1.1653x over previous
import functools

import jax
import jax.numpy as jnp
from jax import lax
from jax.experimental import pallas as pl
from jax.experimental.pallas import tpu as pltpu
from jax.experimental.pallas import tpu_sc as plsc

B, D, K = 8192, 512, 1024

BM = 1024
NB = B // BM
KB = K // NB

NC, NS = 2, 16
NW = NC * NS
BPW = B // NW
CHUNK = 32
NCH = BPW // CHUNK
NBUF = 6


def _argmax_body(x_ref, we_ref, be_ref, wd_ref, bd_ref, idx_ref, tab_ref):
    logits = jnp.dot(x_ref[...], we_ref[...],
                     preferred_element_type=jnp.float32)
    logits = logits + be_ref[...]
    idx = jnp.argmax(logits, axis=1).astype(jnp.int32)
    idx_ref[...] = idx.reshape(idx_ref.shape)
    tab_ref[...] = wd_ref[...] + bd_ref[...]


_argmax_call = pl.pallas_call(
    _argmax_body,
    grid=(NB,),
    in_specs=[
        pl.BlockSpec((BM, D), lambda i: (i, 0)),
        pl.BlockSpec((D, K), lambda i: (0, 0)),
        pl.BlockSpec((1, K), lambda i: (0, 0)),
        pl.BlockSpec((KB, D), lambda i: (i, 0)),
        pl.BlockSpec((1, D), lambda i: (0, 0)),
    ],
    out_specs=[
        pl.BlockSpec((1, 1, BM), lambda i: (i, 0, 0)),
        pl.BlockSpec((KB, D), lambda i: (i, 0)),
    ],
    out_shape=[
        jax.ShapeDtypeStruct((NB, 1, BM), jnp.int32),
        jax.ShapeDtypeStruct((K, D), jnp.float32),
    ],
)


@functools.cache
def _sc_gather_call():
    @functools.partial(
        pl.kernel,
        out_type=jax.ShapeDtypeStruct((B, D), jnp.float32),
        mesh=plsc.VectorSubcoreMesh(core_axis_name="c", subcore_axis_name="s",
                                    num_cores=NC, num_subcores=NS),
        scratch_types=(
            [pltpu.VMEM((BPW,), jnp.int32)]
            + [pltpu.VMEM((CHUNK, D), jnp.float32) for _ in range(NBUF)]
            + [pltpu.SemaphoreType.DMA for _ in range(2 * NBUF)]
        ),
    )
    def _sc_gather(tab_hbm, idx_hbm, out_hbm, idx_v, *bufs_and_sems):
        rows = bufs_and_sems[:NBUF]
        gsem = bufs_and_sems[NBUF:2 * NBUF]
        wsem = bufs_and_sems[2 * NBUF:]
        wid = lax.axis_index("s") * NC + lax.axis_index("c")
        base = wid * BPW
        pltpu.sync_copy(idx_hbm.at[pl.ds(base, BPW)], idx_v)

        def start_gather(c):
            b = c % NBUF
            return pltpu.async_copy(
                tab_hbm.at[idx_v.at[pl.ds(c * CHUNK, CHUNK)]], rows[b], gsem[b])

        gathers = {c: start_gather(c) for c in range(min(NBUF, NCH))}
        writes = {}
        for c in range(NCH):
            b = c % NBUF
            gathers[c].wait()
            writes[c] = pltpu.async_copy(
                rows[b], out_hbm.at[pl.ds(base + c * CHUNK, CHUNK)], wsem[b])
            nxt = c + NBUF
            if nxt < NCH:
                writes[nxt - NBUF].wait()
                gathers[nxt] = start_gather(nxt)
        for c in range(max(0, NCH - NBUF), NCH):
            writes[c].wait()

    return _sc_gather


def kernel(x, W_enc, b_enc, W_dec, b_dec):
    idx3, tab = _argmax_call(x, W_enc, b_enc.reshape(1, K),
                             W_dec, b_dec.reshape(1, D))
    return _sc_gather_call()(tab, idx3.reshape(B))
